# baseline (speedup 1.0000x reference)
import functools
import jax
import jax.numpy as jnp
from jax.experimental import pallas as pl

N = 10000
E = 160000
D = 256
CONV_H = 256
LIN_H = 128
NUM_CLASSES = 10
NUM_GRAPHS = 64

ROW_BLK = 2000


def _mm_kernel(x_ref, w_ref, o_ref):
    o_ref[...] = jnp.dot(x_ref[...], w_ref[...], preferred_element_type=jnp.float32)


def _matmul(x, w):
    m, k = x.shape
    _, n = w.shape
    blk = ROW_BLK if m % ROW_BLK == 0 else m
    return pl.pallas_call(
        _mm_kernel,
        grid=(m // blk,),
        in_specs=[
            pl.BlockSpec((blk, k), lambda i: (i, 0)),
            pl.BlockSpec((k, n), lambda i: (0, 0)),
        ],
        out_specs=pl.BlockSpec((blk, n), lambda i: (i, 0)),
        out_shape=jax.ShapeDtypeStruct((m, n), jnp.float32),
    )(x, w)


def _gcn_conv(x, src, dst, ew, W, b, norm):
    xw = _matmul(x, W)
    msg = norm[:, None] * jnp.take(xw, src, axis=0)
    out = jnp.zeros((N, W.shape[1]), dtype=x.dtype).at[dst].add(msg)
    return out + b


def kernel(x, edge_index, edge_weight, batch, W1, b1, W2, b2, W3, b3, lin1_W, lin1_b, lin2_W, lin2_b):
    ew = jnp.ravel(edge_weight)
    src = edge_index[0].astype(jnp.int32)
    dst = edge_index[1].astype(jnp.int32)
    loop = jnp.arange(N, dtype=jnp.int32)
    s = jnp.concatenate([src, loop])
    d = jnp.concatenate([dst, loop])
    w = jnp.concatenate([ew, jnp.ones((N,), dtype=ew.dtype)])
    deg = jnp.zeros((N,), dtype=x.dtype).at[d].add(w)
    dinv = jnp.where(deg > 0, deg ** -0.5, 0.0)
    norm = dinv[s] * w * dinv[d]

    h = jax.nn.relu(_gcn_conv(x, s, d, w, W1, b1, norm))
    h = jax.nn.relu(_gcn_conv(h, s, d, w, W2, b2, norm))
    h = jax.nn.relu(_gcn_conv(h, s, d, w, W3, b3, norm))
    pooled = jax.ops.segment_max(h, batch, num_segments=NUM_GRAPHS)
    pooled = jnp.where(jnp.isneginf(pooled), 0.0, pooled)
    out = jax.nn.relu(_matmul(pooled, lin1_W) + lin1_b)
    out = _matmul(out, lin2_W) + lin2_b
    return out
